# Initial kernel scaffold; baseline (speedup 1.0000x reference)
import jax
import jax.numpy as jnp
from jax.experimental import pallas as pl


def kernel(x, edge_index, edge_weight, batch, W1, b1, W2, b2, W3, b3, Wl1, bl1, Wl2, bl2, Wl3, bl3):
    raise NotImplementedError("write your pallas kernel here")


# speedup vs baseline: 3.4765x; 3.4765x over previous
import functools

import jax
import jax.numpy as jnp
from jax import lax
from jax.experimental import pallas as pl
from jax.experimental.pallas import tpu as pltpu
from jax.experimental.pallas import tpu_sc as plsc

NC, NS, LANES = 2, 16, 16
NW = NC * NS
CHUNK = 128
NBUF = 2
NPASS = 4
G = 64


def _make_spmm(N, D, NCH):
    SLAB = (N // (8 * NS)) * 8
    TAIL = N - SLAB * NS
    n_full, rem = divmod(SLAB, CHUNK)
    PCH = NCH // NPASS
    NT = PCH // NBUF
    mesh = plsc.VectorSubcoreMesh(core_axis_name="c", subcore_axis_name="s")

    @functools.partial(
        pl.kernel,
        out_type=jax.ShapeDtypeStruct((NC, N, D), jnp.float32),
        mesh=mesh,
        scratch_types=[
            pltpu.VMEM((PCH, CHUNK), jnp.int32),
            pltpu.VMEM((PCH, CHUNK), jnp.int32),
            pltpu.VMEM((PCH, CHUNK), jnp.float32),
            pltpu.VMEM((CHUNK, D), jnp.float32),
            pltpu.VMEM((CHUNK, D), jnp.float32),
            pltpu.SemaphoreType.DMA((NBUF,)),
            pltpu.SemaphoreType.DMA((NBUF,)),
            pltpu.VMEM_SHARED((N, D), jnp.float32),
        ],
    )
    def spmm(h_hbm, srcT, dstT, wT, out_hbm,
             src_v, dst_v, w_v, b0, b1, gsem, ssem, agg):
        bufs = (b0, b1)
        c = lax.axis_index("c")
        s = lax.axis_index("s")
        wid = c * NS + s

        zv = jnp.zeros((LANES,), jnp.float32)

        def zrow(e, carry):
            for k in range(D // LANES):
                b0[e, pl.ds(k * LANES, LANES)] = zv
            return carry

        lax.fori_loop(0, CHUNK, zrow, 0)
        base = s * SLAB
        for i in range(n_full):
            pltpu.sync_copy(b0, agg.at[pl.ds(base + i * CHUNK, CHUNK)])
        if rem:
            pltpu.sync_copy(b0.at[pl.ds(0, rem)],
                            agg.at[pl.ds(base + n_full * CHUNK, rem)])
        if TAIL:
            @pl.when(s == NS - 1)
            def _():
                pltpu.sync_copy(b0.at[pl.ds(0, TAIL)],
                                agg.at[pl.ds(NS * SLAB, TAIL)])

        def issue_gather(j, b):
            pltpu.async_copy(h_hbm.at[src_v.at[j]], bufs[b], gsem.at[b])

        def wait_gather(b):
            pltpu.make_async_copy(h_hbm.at[pl.ds(0, CHUNK)], bufs[b],
                                  gsem.at[b]).wait()

        def issue_scatter(j, b):
            pltpu.async_copy(bufs[b], agg.at[dst_v.at[j]], ssem.at[b], add=True)

        def wait_scatter(b):
            pltpu.make_async_copy(h_hbm.at[pl.ds(0, CHUNK)], bufs[b],
                                  ssem.at[b]).wait()

        plsc.subcore_barrier()

        for p in range(NPASS):
            pltpu.sync_copy(srcT.at[wid * NPASS + p], src_v)
            pltpu.sync_copy(dstT.at[wid * NPASS + p], dst_v)
            pltpu.sync_copy(wT.at[wid * NPASS + p], w_v)
            issue_gather(0, 0)

            def outer(t, carry):
                for b in range(NBUF):
                    j = t * NBUF + b
                    wait_gather(b)
                    jn = j + 1
                    bn = (b + 1) % NBUF

                    @pl.when(jn < PCH)
                    def _():
                        @pl.when(j >= 1)
                        def _():
                            wait_scatter(bn)
                        issue_gather(jn, bn)

                    def srow(g, c2):
                        wvec = w_v[j, pl.ds(g * LANES, LANES)]
                        for u in range(LANES):
                            e = g * LANES + u
                            wv = jnp.full((LANES,), wvec[u], jnp.float32)
                            for k in range(D // LANES):
                                sl = pl.ds(k * LANES, LANES)
                                bufs[b][e, sl] = bufs[b][e, sl] * wv
                        return c2

                    lax.fori_loop(0, CHUNK // LANES, srow, 0)
                    issue_scatter(j, b)
                return carry

            lax.fori_loop(0, NT, outer, 0)
            for b in range(NBUF):
                wait_scatter(b)
        plsc.subcore_barrier()
        pltpu.sync_copy(agg.at[pl.ds(base, SLAB)],
                        out_hbm.at[c, pl.ds(base, SLAB)])
        if TAIL:
            @pl.when(s == NS - 1)
            def _():
                pltpu.sync_copy(agg.at[pl.ds(NS * SLAB, TAIL)],
                                out_hbm.at[c, pl.ds(NS * SLAB, TAIL)])

    return spmm


def _dense_layer(p, W, b2d):
    _, N, D = p.shape
    H = W.shape[1]
    RB = 1000

    def body(p_ref, W_ref, b_ref, o_ref):
        t = p_ref[0] + p_ref[1]
        y = jnp.dot(t, W_ref[...], preferred_element_type=jnp.float32)
        o_ref[...] = jnp.maximum(y + b_ref[...], 0.0)

    return pl.pallas_call(
        body,
        grid=(N // RB,),
        in_specs=[
            pl.BlockSpec((2, RB, D), lambda i: (0, i, 0)),
            pl.BlockSpec((D, H), lambda i: (0, 0)),
            pl.BlockSpec((1, H), lambda i: (0, 0)),
        ],
        out_specs=pl.BlockSpec((RB, H), lambda i: (i, 0)),
        out_shape=jax.ShapeDtypeStruct((N, H), jnp.float32),
    )(p, W, b2d)


def _head(p3, W3, b3d, batch2d, Wl1, bl1d, Wl2, bl2d, Wl3, bl3d):
    _, N, D = p3.shape
    H = W3.shape[1]
    H1 = Wl1.shape[1]
    H2 = Wl2.shape[1]
    RB = 1000
    nb = N // RB

    def body(p_ref, W3_ref, b3_ref, bt_ref, Wl1_ref, bl1_ref, Wl2_ref,
             bl2_ref, Wl3_ref, bl3_ref, o_ref, sums, counts):
        i = pl.program_id(0)

        @pl.when(i == 0)
        def _():
            sums[...] = jnp.zeros_like(sums)
            counts[...] = jnp.zeros_like(counts)

        t = p_ref[0] + p_ref[1]
        h = jnp.maximum(
            jnp.dot(t, W3_ref[...], preferred_element_type=jnp.float32)
            + b3_ref[...], 0.0)
        oh = (bt_ref[...] == lax.broadcasted_iota(jnp.int32, (RB, G), 1)
              ).astype(jnp.float32)
        dn = (((0,), (0,)), ((), ()))
        sums[...] += lax.dot_general(oh, h, dn,
                                     preferred_element_type=jnp.float32)
        counts[...] += lax.dot_general(
            oh, jnp.ones((RB, 1), jnp.float32), dn,
            preferred_element_type=jnp.float32)

        @pl.when(i == nb - 1)
        def _():
            pooled = sums[...] / jnp.maximum(counts[...], 1.0)
            y = jnp.maximum(
                jnp.dot(pooled, Wl1_ref[...],
                        preferred_element_type=jnp.float32) + bl1_ref[...], 0.0)
            y = jnp.maximum(
                jnp.dot(y, Wl2_ref[...],
                        preferred_element_type=jnp.float32) + bl2_ref[...], 0.0)
            o_ref[...] = jnp.dot(
                y, Wl3_ref[...],
                preferred_element_type=jnp.float32) + bl3_ref[...]

    return pl.pallas_call(
        body,
        grid=(nb,),
        in_specs=[
            pl.BlockSpec((2, RB, D), lambda i: (0, i, 0)),
            pl.BlockSpec((D, H), lambda i: (0, 0)),
            pl.BlockSpec((1, H), lambda i: (0, 0)),
            pl.BlockSpec((RB, 1), lambda i: (i, 0)),
            pl.BlockSpec((H, H1), lambda i: (0, 0)),
            pl.BlockSpec((1, H1), lambda i: (0, 0)),
            pl.BlockSpec((H1, H2), lambda i: (0, 0)),
            pl.BlockSpec((1, H2), lambda i: (0, 0)),
            pl.BlockSpec((H2, 1), lambda i: (0, 0)),
            pl.BlockSpec((1, 1), lambda i: (0, 0)),
        ],
        out_specs=pl.BlockSpec((G, 1), lambda i: (0, 0)),
        out_shape=jax.ShapeDtypeStruct((G, 1), jnp.float32),
        scratch_shapes=[
            pltpu.VMEM((G, H), jnp.float32),
            pltpu.VMEM((G, 1), jnp.float32),
        ],
    )(p3, W3, b3d, batch2d, Wl1, bl1d, Wl2, bl2d, Wl3, bl3d)


def kernel(x, edge_index, edge_weight, batch,
           W1, b1, W2, b2, W3, b3,
           Wl1, bl1, Wl2, bl2, Wl3, bl3):
    N, D = x.shape
    E = edge_weight.shape[0]
    nch = -(-E // (NW * CHUNK))
    step = NPASS * NBUF
    NCH = -(-nch // step) * step
    EP = NW * NCH * CHUNK
    pad = EP - E
    src = edge_index[0]
    dst = edge_index[1]
    w = edge_weight
    if pad:
        zi = jnp.zeros((pad,), jnp.int32)
        src = jnp.concatenate([src, zi])
        dst = jnp.concatenate([dst, zi])
        w = jnp.concatenate([w, jnp.zeros((pad,), jnp.float32)])
    PCH = NCH // NPASS
    srcT = src.reshape(NW * NPASS, PCH, CHUNK)
    dstT = dst.reshape(NW * NPASS, PCH, CHUNK)
    wT = w.reshape(NW * NPASS, PCH, CHUNK)

    spmm = _make_spmm(N, D, NCH)
    p1 = spmm(x, srcT, dstT, wT)
    h1 = _dense_layer(p1, W1, b1.reshape(1, -1))
    p2 = spmm(h1, srcT, dstT, wT)
    h2 = _dense_layer(p2, W2, b2.reshape(1, -1))
    p3 = spmm(h2, srcT, dstT, wT)
    return _head(p3, W3, b3.reshape(1, -1), batch.reshape(N, 1),
                 Wl1, bl1.reshape(1, -1), Wl2, bl2.reshape(1, -1),
                 Wl3, bl3.reshape(1, -1))


# speedup vs baseline: 3.5250x; 1.0140x over previous
import functools

import jax
import jax.numpy as jnp
from jax import lax
from jax.experimental import pallas as pl
from jax.experimental.pallas import tpu as pltpu
from jax.experimental.pallas import tpu_sc as plsc

NC, NS, LANES = 2, 16, 16
NW = NC * NS
CHUNK = 128
NBUF = 2
NPASS = 4
G = 64


def _make_spmm(N, D, NCH):
    SLAB = (N // (8 * NS)) * 8
    TAIL = N - SLAB * NS
    n_full, rem = divmod(SLAB, CHUNK)
    PCH = NCH // NPASS
    NT = PCH // NBUF
    mesh = plsc.VectorSubcoreMesh(core_axis_name="c", subcore_axis_name="s")

    @functools.partial(
        pl.kernel,
        out_type=jax.ShapeDtypeStruct((NC, N, D), jnp.float32),
        mesh=mesh,
        scratch_types=[
            pltpu.VMEM((PCH, CHUNK), jnp.int32),
            pltpu.VMEM((PCH, CHUNK), jnp.int32),
            pltpu.VMEM((PCH, CHUNK), jnp.float32),
            pltpu.VMEM((CHUNK, D), jnp.float32),
            pltpu.VMEM((CHUNK, D), jnp.float32),
            pltpu.SemaphoreType.DMA((NBUF,)),
            pltpu.SemaphoreType.DMA((NBUF,)),
            pltpu.VMEM_SHARED((N, D), jnp.float32),
        ],
    )
    def spmm(h_hbm, srcT, dstT, wT, out_hbm,
             src_v, dst_v, w_v, b0, b1, gsem, ssem, agg):
        bufs = (b0, b1)
        c = lax.axis_index("c")
        s = lax.axis_index("s")
        wid = c * NS + s

        zv = jnp.zeros((LANES,), jnp.float32)

        def zrow(e, carry):
            for k in range(D // LANES):
                b0[e, pl.ds(k * LANES, LANES)] = zv
            return carry

        lax.fori_loop(0, CHUNK, zrow, 0)
        base = s * SLAB
        for i in range(n_full):
            pltpu.sync_copy(b0, agg.at[pl.ds(base + i * CHUNK, CHUNK)])
        if rem:
            pltpu.sync_copy(b0.at[pl.ds(0, rem)],
                            agg.at[pl.ds(base + n_full * CHUNK, rem)])
        if TAIL:
            @pl.when(s == NS - 1)
            def _():
                pltpu.sync_copy(b0.at[pl.ds(0, TAIL)],
                                agg.at[pl.ds(NS * SLAB, TAIL)])

        def issue_gather(j, b):
            pltpu.async_copy(h_hbm.at[src_v.at[j]], bufs[b], gsem.at[b])

        def wait_gather(b):
            pltpu.make_async_copy(h_hbm.at[pl.ds(0, CHUNK)], bufs[b],
                                  gsem.at[b]).wait()

        DIAG_NO_SCATTER = True

        def issue_scatter(j, b):
            if DIAG_NO_SCATTER:
                return
            pltpu.async_copy(bufs[b], agg.at[dst_v.at[j]], ssem.at[b], add=True)

        def wait_scatter(b):
            if DIAG_NO_SCATTER:
                return
            pltpu.make_async_copy(h_hbm.at[pl.ds(0, CHUNK)], bufs[b],
                                  ssem.at[b]).wait()

        plsc.subcore_barrier()

        for p in range(NPASS):
            pltpu.sync_copy(srcT.at[wid * NPASS + p], src_v)
            pltpu.sync_copy(dstT.at[wid * NPASS + p], dst_v)
            pltpu.sync_copy(wT.at[wid * NPASS + p], w_v)
            issue_gather(0, 0)

            def outer(t, carry):
                for b in range(NBUF):
                    j = t * NBUF + b
                    wait_gather(b)
                    jn = j + 1
                    bn = (b + 1) % NBUF

                    @pl.when(jn < PCH)
                    def _():
                        @pl.when(j >= 1)
                        def _():
                            wait_scatter(bn)
                        issue_gather(jn, bn)

                    def srow(g, c2):
                        wvec = w_v[j, pl.ds(g * LANES, LANES)]
                        for u in range(LANES):
                            e = g * LANES + u
                            wv = jnp.full((LANES,), wvec[u], jnp.float32)
                            for k in range(D // LANES):
                                sl = pl.ds(k * LANES, LANES)
                                bufs[b][e, sl] = bufs[b][e, sl] * wv
                        return c2

                    if not DIAG_NO_SCATTER:
                        lax.fori_loop(0, CHUNK // LANES, srow, 0)
                    issue_scatter(j, b)
                return carry

            lax.fori_loop(0, NT, outer, 0)
            for b in range(NBUF):
                wait_scatter(b)
        plsc.subcore_barrier()
        pltpu.sync_copy(agg.at[pl.ds(base, SLAB)],
                        out_hbm.at[c, pl.ds(base, SLAB)])
        if TAIL:
            @pl.when(s == NS - 1)
            def _():
                pltpu.sync_copy(agg.at[pl.ds(NS * SLAB, TAIL)],
                                out_hbm.at[c, pl.ds(NS * SLAB, TAIL)])

    return spmm


def _dense_layer(p, W, b2d):
    _, N, D = p.shape
    H = W.shape[1]
    RB = 1000

    def body(p_ref, W_ref, b_ref, o_ref):
        t = p_ref[0] + p_ref[1]
        y = jnp.dot(t, W_ref[...], preferred_element_type=jnp.float32)
        o_ref[...] = jnp.maximum(y + b_ref[...], 0.0)

    return pl.pallas_call(
        body,
        grid=(N // RB,),
        in_specs=[
            pl.BlockSpec((2, RB, D), lambda i: (0, i, 0)),
            pl.BlockSpec((D, H), lambda i: (0, 0)),
            pl.BlockSpec((1, H), lambda i: (0, 0)),
        ],
        out_specs=pl.BlockSpec((RB, H), lambda i: (i, 0)),
        out_shape=jax.ShapeDtypeStruct((N, H), jnp.float32),
    )(p, W, b2d)


def _head(p3, W3, b3d, batch2d, Wl1, bl1d, Wl2, bl2d, Wl3, bl3d):
    _, N, D = p3.shape
    H = W3.shape[1]
    H1 = Wl1.shape[1]
    H2 = Wl2.shape[1]
    RB = 1000
    nb = N // RB

    def body(p_ref, W3_ref, b3_ref, bt_ref, Wl1_ref, bl1_ref, Wl2_ref,
             bl2_ref, Wl3_ref, bl3_ref, o_ref, sums, counts):
        i = pl.program_id(0)

        @pl.when(i == 0)
        def _():
            sums[...] = jnp.zeros_like(sums)
            counts[...] = jnp.zeros_like(counts)

        t = p_ref[0] + p_ref[1]
        h = jnp.maximum(
            jnp.dot(t, W3_ref[...], preferred_element_type=jnp.float32)
            + b3_ref[...], 0.0)
        oh = (bt_ref[...] == lax.broadcasted_iota(jnp.int32, (RB, G), 1)
              ).astype(jnp.float32)
        dn = (((0,), (0,)), ((), ()))
        sums[...] += lax.dot_general(oh, h, dn,
                                     preferred_element_type=jnp.float32)
        counts[...] += lax.dot_general(
            oh, jnp.ones((RB, 1), jnp.float32), dn,
            preferred_element_type=jnp.float32)

        @pl.when(i == nb - 1)
        def _():
            pooled = sums[...] / jnp.maximum(counts[...], 1.0)
            y = jnp.maximum(
                jnp.dot(pooled, Wl1_ref[...],
                        preferred_element_type=jnp.float32) + bl1_ref[...], 0.0)
            y = jnp.maximum(
                jnp.dot(y, Wl2_ref[...],
                        preferred_element_type=jnp.float32) + bl2_ref[...], 0.0)
            o_ref[...] = jnp.dot(
                y, Wl3_ref[...],
                preferred_element_type=jnp.float32) + bl3_ref[...]

    return pl.pallas_call(
        body,
        grid=(nb,),
        in_specs=[
            pl.BlockSpec((2, RB, D), lambda i: (0, i, 0)),
            pl.BlockSpec((D, H), lambda i: (0, 0)),
            pl.BlockSpec((1, H), lambda i: (0, 0)),
            pl.BlockSpec((RB, 1), lambda i: (i, 0)),
            pl.BlockSpec((H, H1), lambda i: (0, 0)),
            pl.BlockSpec((1, H1), lambda i: (0, 0)),
            pl.BlockSpec((H1, H2), lambda i: (0, 0)),
            pl.BlockSpec((1, H2), lambda i: (0, 0)),
            pl.BlockSpec((H2, 1), lambda i: (0, 0)),
            pl.BlockSpec((1, 1), lambda i: (0, 0)),
        ],
        out_specs=pl.BlockSpec((G, 1), lambda i: (0, 0)),
        out_shape=jax.ShapeDtypeStruct((G, 1), jnp.float32),
        scratch_shapes=[
            pltpu.VMEM((G, H), jnp.float32),
            pltpu.VMEM((G, 1), jnp.float32),
        ],
    )(p3, W3, b3d, batch2d, Wl1, bl1d, Wl2, bl2d, Wl3, bl3d)


def kernel(x, edge_index, edge_weight, batch,
           W1, b1, W2, b2, W3, b3,
           Wl1, bl1, Wl2, bl2, Wl3, bl3):
    N, D = x.shape
    E = edge_weight.shape[0]
    nch = -(-E // (NW * CHUNK))
    step = NPASS * NBUF
    NCH = -(-nch // step) * step
    EP = NW * NCH * CHUNK
    pad = EP - E
    src = edge_index[0]
    dst = edge_index[1]
    w = edge_weight
    if pad:
        zi = jnp.zeros((pad,), jnp.int32)
        src = jnp.concatenate([src, zi])
        dst = jnp.concatenate([dst, zi])
        w = jnp.concatenate([w, jnp.zeros((pad,), jnp.float32)])
    PCH = NCH // NPASS
    srcT = src.reshape(NW * NPASS, PCH, CHUNK)
    dstT = dst.reshape(NW * NPASS, PCH, CHUNK)
    wT = w.reshape(NW * NPASS, PCH, CHUNK)

    spmm = _make_spmm(N, D, NCH)
    p1 = spmm(x, srcT, dstT, wT)
    h1 = _dense_layer(p1, W1, b1.reshape(1, -1))
    p2 = spmm(h1, srcT, dstT, wT)
    h2 = _dense_layer(p2, W2, b2.reshape(1, -1))
    p3 = spmm(h2, srcT, dstT, wT)
    return _head(p3, W3, b3.reshape(1, -1), batch.reshape(N, 1),
                 Wl1, bl1.reshape(1, -1), Wl2, bl2.reshape(1, -1),
                 Wl3, bl3.reshape(1, -1))


# speedup vs baseline: 3.7508x; 1.0641x over previous
import functools

import jax
import jax.numpy as jnp
from jax import lax
from jax.experimental import pallas as pl
from jax.experimental.pallas import tpu as pltpu
from jax.experimental.pallas import tpu_sc as plsc

NC, NS, LANES = 2, 16, 16
NW = NC * NS
CHUNK = 64
NBUF = 4
NPASS = 8
G = 64


def _make_spmm(N, D, NCH):
    SLAB = (N // (8 * NS)) * 8
    TAIL = N - SLAB * NS
    n_full, rem = divmod(SLAB, CHUNK)
    PCH = NCH // NPASS
    NT = PCH // NBUF
    mesh = plsc.VectorSubcoreMesh(core_axis_name="c", subcore_axis_name="s")

    @functools.partial(
        pl.kernel,
        out_type=jax.ShapeDtypeStruct((NC, N, D), jnp.float32),
        mesh=mesh,
        scratch_types=[
            pltpu.VMEM((PCH, CHUNK), jnp.int32),
            pltpu.VMEM((PCH, CHUNK), jnp.int32),
            pltpu.VMEM((PCH, CHUNK), jnp.float32),
            pltpu.VMEM((CHUNK, D), jnp.float32),
            pltpu.VMEM((CHUNK, D), jnp.float32),
            pltpu.VMEM((CHUNK, D), jnp.float32),
            pltpu.VMEM((CHUNK, D), jnp.float32),
            pltpu.SemaphoreType.DMA((NBUF,)),
            pltpu.SemaphoreType.DMA((NBUF,)),
            pltpu.VMEM_SHARED((N, D), jnp.float32),
        ],
    )
    def spmm(h_hbm, srcT, dstT, wT, out_hbm,
             src_v, dst_v, w_v, b0, b1, b2, b3, gsem, ssem, agg):
        bufs = (b0, b1, b2, b3)
        c = lax.axis_index("c")
        s = lax.axis_index("s")
        wid = c * NS + s

        zv = jnp.zeros((LANES,), jnp.float32)

        def zrow(e, carry):
            for k in range(D // LANES):
                b0[e, pl.ds(k * LANES, LANES)] = zv
            return carry

        lax.fori_loop(0, CHUNK, zrow, 0)
        base = s * SLAB
        for i in range(n_full):
            pltpu.sync_copy(b0, agg.at[pl.ds(base + i * CHUNK, CHUNK)])
        if rem:
            pltpu.sync_copy(b0.at[pl.ds(0, rem)],
                            agg.at[pl.ds(base + n_full * CHUNK, rem)])
        if TAIL:
            @pl.when(s == NS - 1)
            def _():
                pltpu.sync_copy(b0.at[pl.ds(0, TAIL)],
                                agg.at[pl.ds(NS * SLAB, TAIL)])

        def issue_gather(j, b):
            pltpu.async_copy(h_hbm.at[src_v.at[j]], bufs[b], gsem.at[b])

        def wait_gather(b):
            pltpu.make_async_copy(h_hbm.at[pl.ds(0, CHUNK)], bufs[b],
                                  gsem.at[b]).wait()

        def issue_scatter(j, b):
            pltpu.async_copy(bufs[b], agg.at[dst_v.at[j]], ssem.at[b], add=True)

        def wait_scatter(b):
            pltpu.make_async_copy(h_hbm.at[pl.ds(0, CHUNK)], bufs[b],
                                  ssem.at[b]).wait()

        plsc.subcore_barrier()

        PRIME = NBUF - 1
        for p in range(NPASS):
            pltpu.sync_copy(srcT.at[wid * NPASS + p], src_v)
            pltpu.sync_copy(dstT.at[wid * NPASS + p], dst_v)
            pltpu.sync_copy(wT.at[wid * NPASS + p], w_v)
            for jj in range(PRIME):
                issue_gather(jj, jj)

            def outer(t, carry):
                for b in range(NBUF):
                    j = t * NBUF + b
                    wait_gather(b)
                    jn = j + PRIME
                    bn = (b + PRIME) % NBUF

                    @pl.when(jn < PCH)
                    def _():
                        @pl.when(j >= 1)
                        def _():
                            wait_scatter(bn)
                        issue_gather(jn, bn)

                    def srow(g, c2):
                        wvec = w_v[j, pl.ds(g * LANES, LANES)]
                        for u in range(LANES):
                            e = g * LANES + u
                            wv = jnp.full((LANES,), wvec[u], jnp.float32)
                            for k in range(D // LANES):
                                sl = pl.ds(k * LANES, LANES)
                                bufs[b][e, sl] = bufs[b][e, sl] * wv
                        return c2

                    lax.fori_loop(0, CHUNK // LANES, srow, 0)
                    issue_scatter(j, b)
                return carry

            lax.fori_loop(0, NT, outer, 0)
            for b in range(NBUF):
                wait_scatter(b)
        plsc.subcore_barrier()
        pltpu.sync_copy(agg.at[pl.ds(base, SLAB)],
                        out_hbm.at[c, pl.ds(base, SLAB)])
        if TAIL:
            @pl.when(s == NS - 1)
            def _():
                pltpu.sync_copy(agg.at[pl.ds(NS * SLAB, TAIL)],
                                out_hbm.at[c, pl.ds(NS * SLAB, TAIL)])

    return spmm


def _dense_layer(p, W, b2d):
    _, N, D = p.shape
    H = W.shape[1]
    RB = 1000

    def body(p_ref, W_ref, b_ref, o_ref):
        t = p_ref[0] + p_ref[1]
        y = jnp.dot(t, W_ref[...], preferred_element_type=jnp.float32)
        o_ref[...] = jnp.maximum(y + b_ref[...], 0.0)

    return pl.pallas_call(
        body,
        grid=(N // RB,),
        in_specs=[
            pl.BlockSpec((2, RB, D), lambda i: (0, i, 0)),
            pl.BlockSpec((D, H), lambda i: (0, 0)),
            pl.BlockSpec((1, H), lambda i: (0, 0)),
        ],
        out_specs=pl.BlockSpec((RB, H), lambda i: (i, 0)),
        out_shape=jax.ShapeDtypeStruct((N, H), jnp.float32),
    )(p, W, b2d)


def _head(p3, W3, b3d, batch2d, Wl1, bl1d, Wl2, bl2d, Wl3, bl3d):
    _, N, D = p3.shape
    H = W3.shape[1]
    H1 = Wl1.shape[1]
    H2 = Wl2.shape[1]
    RB = 1000
    nb = N // RB

    def body(p_ref, W3_ref, b3_ref, bt_ref, Wl1_ref, bl1_ref, Wl2_ref,
             bl2_ref, Wl3_ref, bl3_ref, o_ref, sums, counts):
        i = pl.program_id(0)

        @pl.when(i == 0)
        def _():
            sums[...] = jnp.zeros_like(sums)
            counts[...] = jnp.zeros_like(counts)

        t = p_ref[0] + p_ref[1]
        h = jnp.maximum(
            jnp.dot(t, W3_ref[...], preferred_element_type=jnp.float32)
            + b3_ref[...], 0.0)
        oh = (bt_ref[...] == lax.broadcasted_iota(jnp.int32, (RB, G), 1)
              ).astype(jnp.float32)
        dn = (((0,), (0,)), ((), ()))
        sums[...] += lax.dot_general(oh, h, dn,
                                     preferred_element_type=jnp.float32)
        counts[...] += lax.dot_general(
            oh, jnp.ones((RB, 1), jnp.float32), dn,
            preferred_element_type=jnp.float32)

        @pl.when(i == nb - 1)
        def _():
            pooled = sums[...] / jnp.maximum(counts[...], 1.0)
            y = jnp.maximum(
                jnp.dot(pooled, Wl1_ref[...],
                        preferred_element_type=jnp.float32) + bl1_ref[...], 0.0)
            y = jnp.maximum(
                jnp.dot(y, Wl2_ref[...],
                        preferred_element_type=jnp.float32) + bl2_ref[...], 0.0)
            o_ref[...] = jnp.dot(
                y, Wl3_ref[...],
                preferred_element_type=jnp.float32) + bl3_ref[...]

    return pl.pallas_call(
        body,
        grid=(nb,),
        in_specs=[
            pl.BlockSpec((2, RB, D), lambda i: (0, i, 0)),
            pl.BlockSpec((D, H), lambda i: (0, 0)),
            pl.BlockSpec((1, H), lambda i: (0, 0)),
            pl.BlockSpec((RB, 1), lambda i: (i, 0)),
            pl.BlockSpec((H, H1), lambda i: (0, 0)),
            pl.BlockSpec((1, H1), lambda i: (0, 0)),
            pl.BlockSpec((H1, H2), lambda i: (0, 0)),
            pl.BlockSpec((1, H2), lambda i: (0, 0)),
            pl.BlockSpec((H2, 1), lambda i: (0, 0)),
            pl.BlockSpec((1, 1), lambda i: (0, 0)),
        ],
        out_specs=pl.BlockSpec((G, 1), lambda i: (0, 0)),
        out_shape=jax.ShapeDtypeStruct((G, 1), jnp.float32),
        scratch_shapes=[
            pltpu.VMEM((G, H), jnp.float32),
            pltpu.VMEM((G, 1), jnp.float32),
        ],
    )(p3, W3, b3d, batch2d, Wl1, bl1d, Wl2, bl2d, Wl3, bl3d)


def kernel(x, edge_index, edge_weight, batch,
           W1, b1, W2, b2, W3, b3,
           Wl1, bl1, Wl2, bl2, Wl3, bl3):
    N, D = x.shape
    E = edge_weight.shape[0]
    nch = -(-E // (NW * CHUNK))
    step = NPASS * NBUF
    NCH = -(-nch // step) * step
    EP = NW * NCH * CHUNK
    pad = EP - E
    src = edge_index[0]
    dst = edge_index[1]
    w = edge_weight
    if pad:
        zi = jnp.zeros((pad,), jnp.int32)
        src = jnp.concatenate([src, zi])
        dst = jnp.concatenate([dst, zi])
        w = jnp.concatenate([w, jnp.zeros((pad,), jnp.float32)])
    PCH = NCH // NPASS
    srcT = src.reshape(NW * NPASS, PCH, CHUNK)
    dstT = dst.reshape(NW * NPASS, PCH, CHUNK)
    wT = w.reshape(NW * NPASS, PCH, CHUNK)

    spmm = _make_spmm(N, D, NCH)
    p1 = spmm(x, srcT, dstT, wT)
    h1 = _dense_layer(p1, W1, b1.reshape(1, -1))
    p2 = spmm(h1, srcT, dstT, wT)
    h2 = _dense_layer(p2, W2, b2.reshape(1, -1))
    p3 = spmm(h2, srcT, dstT, wT)
    return _head(p3, W3, b3.reshape(1, -1), batch.reshape(N, 1),
                 Wl1, bl1.reshape(1, -1), Wl2, bl2.reshape(1, -1),
                 Wl3, bl3.reshape(1, -1))
